# Initial kernel scaffold; baseline (speedup 1.0000x reference)
import jax
import jax.numpy as jnp
from jax.experimental import pallas as pl


def kernel(input, target, X):
    raise NotImplementedError("write your pallas kernel here")


# speedup vs baseline: 69.3547x; 69.3547x over previous
import jax
import jax.numpy as jnp
from jax import lax
from jax.experimental import pallas as pl
from jax.experimental.pallas import tpu as pltpu

MARGIN_ = 0.5
C_ = 256
K_ = 16
B_ = 64
E_ = 4096


def _body(preds_ref, target_ref, x_ref, out_ref, d_scr, n_scr):
    f32 = jnp.float32
    preds = preds_ref[...]
    t = target_ref[...]
    ypos = t == 1

    iota_bc_row = lax.broadcasted_iota(jnp.int32, (B_, C_), 0)
    iota_cc_r = lax.broadcasted_iota(jnp.int32, (C_, C_), 0)
    iota_cc_c = lax.broadcasted_iota(jnp.int32, (C_, C_), 1)

    tf = t.astype(f32)
    h_row = jnp.sum(tf, axis=0, keepdims=True)
    ones_b1 = jnp.ones((B_, 1), f32)
    h_col = lax.dot_general(tf, ones_b1, (((0,), (0,)), ((), ())),
                             precision=lax.Precision.HIGHEST)

    before = (h_col < h_row) | ((h_col == h_row) & (iota_cc_r <= iota_cc_c))
    s_cum = jnp.sum(h_col * before.astype(f32), axis=0, keepdims=True)
    minority = (s_cum <= 0.5 * C_) & (h_row > 1.0)

    n_c = h_row
    kp = jnp.minimum(n_c - 1.0, float(K_))
    kn = jnp.minimum(float(B_) - n_c, float(K_))
    class_valid = minority & (n_c < float(B_))
    anchor_valid = class_valid & ypos

    count = jnp.sum(jnp.where(class_valid, n_c * kp * kn, 0.0), keepdims=True)

    x = x_ref[...]
    ones_1e = jnp.ones((1, E_), f32)

    def dist_row(j, _):
        row = x_ref[pl.ds(j, 1), :]
        diff = jnp.abs(x - row)
        d_row = lax.dot_general(ones_1e, diff, (((1,), (1,)), ((), ())),
                             precision=lax.Precision.HIGHEST)
        d_scr[pl.ds(j, 1), :] = d_row
        return 0

    lax.fori_loop(0, B_, dist_row, 0, unroll=2)
    dmat = d_scr[...]

    neg_inf = f32(-jnp.inf)
    cur = jnp.where(ypos, neg_inf, preds)
    for k in range(K_):
        v = jnp.max(cur, axis=0, keepdims=True)
        idx = jnp.max(jnp.where(cur == v, iota_bc_row, -1), axis=0, keepdims=True)
        onehot = iota_bc_row == idx
        nk = lax.dot_general(dmat, onehot.astype(f32),
                             (((1,), (0,)), ((), ())),
                             precision=lax.Precision.HIGHEST)
        n_scr[pl.ds(k * B_, B_), :] = nk
        cur = jnp.where(onehot, neg_inf, cur)

    pos_inf = f32(jnp.inf)
    cur = jnp.where(ypos, preds, pos_inf)
    bseen = jnp.zeros((B_, C_), jnp.bool_)
    acc = jnp.zeros((B_, C_), f32)
    for j in range(K_ + 1):
        v = jnp.min(cur, axis=0, keepdims=True)
        idx = jnp.min(jnp.where(cur == v, iota_bc_row, B_ + 9), axis=0, keepdims=True)
        onehot = iota_bc_row == idx
        pj = lax.dot_general(dmat, onehot.astype(f32),
                             (((1,), (0,)), ((), ())),
                             precision=lax.Precision.HIGHEST)
        rank = f32(j) - bseen.astype(f32)
        validp = anchor_valid & (~onehot) & (rank < kp)
        bseen = bseen | onehot
        cur = jnp.where(onehot, pos_inf, cur)

        pj_valid = jnp.where(validp, pj, neg_inf)

        def neg_term(k, a):
            nk = n_scr[pl.ds(k * B_, B_), :]
            term = jnp.maximum(pj_valid - nk + MARGIN_, 0.0)
            term = jnp.where(k.astype(f32) < kn, term, 0.0)
            return a + term

        acc = lax.fori_loop(0, K_, neg_term, acc, unroll=4)

    total = jnp.sum(acc, keepdims=True)
    out_ref[...] = jnp.where(count > 0.0, total / count, 0.0)


@jax.jit
def kernel(input, target, X):
    out = pl.pallas_call(
        _body,
        out_shape=jax.ShapeDtypeStruct((1, 1), jnp.float32),
        scratch_shapes=[
            pltpu.VMEM((B_, B_), jnp.float32),
            pltpu.VMEM((K_ * B_, C_), jnp.float32),
        ],
    )(input, target, X)
    return out[0, 0]
